# Initial kernel scaffold; baseline (speedup 1.0000x reference)
import jax
import jax.numpy as jnp
from jax.experimental import pallas as pl


def kernel(x, h, J, edge_idx_i, edge_idx_j):
    raise NotImplementedError("write your pallas kernel here")


# speedup vs baseline: 4.9336x; 4.9336x over previous
import functools

import jax
import jax.numpy as jnp
from jax import lax
from jax.experimental import pallas as pl
from jax.experimental.pallas import tpu as pltpu
from jax.experimental.pallas import tpu_sc as plsc

NUM_NODES = 50000
NUM_EDGES = 1600000
BATCH = 32

NC = 2
NS = 16
L = 16

CHUNK = 3200
N_CHUNKS = NUM_EDGES // CHUNK
G_PER_CHUNK = CHUNK // L
H_GROUPS = NUM_NODES // L


def _sc_body(x_hbm, h_hbm, j_hbm, ei_hbm, ej_hbm, out_hbm,
             xb, hv, eic, ejc, jc, accv):
    cid = lax.axis_index("c")
    sid = lax.axis_index("s")
    b = sid * NC + cid

    pltpu.sync_copy(x_hbm.at[b], xb)
    pltpu.sync_copy(h_hbm, hv)

    def h_group(g, acc):
        return acc + xb[pl.ds(g * L, L)] * hv[pl.ds(g * L, L)]

    acc = lax.fori_loop(0, H_GROUPS, h_group, jnp.zeros((L,), jnp.float32))

    def chunk_body(ci, acc):
        base = ci * CHUNK
        pltpu.sync_copy(ei_hbm.at[pl.ds(base, CHUNK)], eic)
        pltpu.sync_copy(ej_hbm.at[pl.ds(base, CHUNK)], ejc)
        pltpu.sync_copy(j_hbm.at[pl.ds(base, CHUNK)], jc)

        def edge_group(g, acc):
            ii = eic[pl.ds(g * L, L)]
            jj = ejc[pl.ds(g * L, L)]
            vi = plsc.load_gather(xb, [ii])
            vj = plsc.load_gather(xb, [jj])
            jv = jc[pl.ds(g * L, L)]
            return acc + jv * (vi * vj)

        return lax.fori_loop(0, G_PER_CHUNK, edge_group, acc)

    acc = lax.fori_loop(0, N_CHUNKS, chunk_body, acc)

    accv[...] = acc
    pltpu.sync_copy(accv, out_hbm.at[b])


@jax.jit
def _sc_energy(x, h, j, ei, ej):
    mesh = plsc.VectorSubcoreMesh(core_axis_name="c", subcore_axis_name="s",
                                  num_cores=NC, num_subcores=NS)
    run = pl.kernel(
        _sc_body,
        out_type=jax.ShapeDtypeStruct((BATCH, L), jnp.float32),
        mesh=mesh,
        compiler_params=pltpu.CompilerParams(needs_layout_passes=False),
        scratch_types=[
            pltpu.VMEM((NUM_NODES,), jnp.float32),
            pltpu.VMEM((NUM_NODES,), jnp.float32),
            pltpu.VMEM((CHUNK,), jnp.int32),
            pltpu.VMEM((CHUNK,), jnp.int32),
            pltpu.VMEM((CHUNK,), jnp.float32),
            pltpu.VMEM((L,), jnp.float32),
        ],
    )
    return run(x, h, j, ei, ej)


def kernel(x, h, J, edge_idx_i, edge_idx_j):
    partials = _sc_energy(x, h, J,
                          edge_idx_i.astype(jnp.int32),
                          edge_idx_j.astype(jnp.int32))
    return partials.sum(axis=-1)


# speedup vs baseline: 11.0393x; 2.2376x over previous
import functools

import jax
import jax.numpy as jnp
from jax import lax
from jax.experimental import pallas as pl
from jax.experimental.pallas import tpu as pltpu
from jax.experimental.pallas import tpu_sc as plsc

NUM_NODES = 50000
NUM_EDGES = 1600000
BATCH = 32

NC = 2
NS = 16
L = 16

CHUNK = 6400
N_CHUNKS = NUM_EDGES // CHUNK
G_PER_CHUNK = CHUNK // L


def _sc_body(x_hbm, j_hbm, ei_hbm, ej_hbm, out_hbm,
             xb, eic, ejc, jc, accv, sem0, sem1):
    cid = lax.axis_index("c")
    sid = lax.axis_index("s")
    b = sid * NC + cid

    pltpu.sync_copy(x_hbm.at[b], xb)

    sems = (sem0, sem1)

    def issue(ci, slot):
        base = lax.rem(ci, N_CHUNKS) * CHUNK
        pltpu.async_copy(ei_hbm.at[pl.ds(base, CHUNK)], eic.at[slot], sems[slot])
        pltpu.async_copy(ej_hbm.at[pl.ds(base, CHUNK)], ejc.at[slot], sems[slot])
        pltpu.async_copy(j_hbm.at[pl.ds(base, CHUNK)], jc.at[slot], sems[slot])

    def drain(slot):
        pltpu.make_async_copy(ei_hbm.at[pl.ds(0, CHUNK)], eic.at[slot], sems[slot]).wait()
        pltpu.make_async_copy(ej_hbm.at[pl.ds(0, CHUNK)], ejc.at[slot], sems[slot]).wait()
        pltpu.make_async_copy(j_hbm.at[pl.ds(0, CHUNK)], jc.at[slot], sems[slot]).wait()

    def compute(slot, acc):
        def edge_group(g, acc):
            ii = eic[slot, pl.ds(g * L, L)]
            jj = ejc[slot, pl.ds(g * L, L)]
            vi = plsc.load_gather(xb, [ii])
            vj = plsc.load_gather(xb, [jj])
            jv = jc[slot, pl.ds(g * L, L)]
            return acc + jv * (vi * vj)

        return lax.fori_loop(0, G_PER_CHUNK, edge_group, acc)

    issue(0, 0)

    def pair_body(p, acc):
        ci = 2 * p
        issue(ci + 1, 1)
        drain(0)
        acc = compute(0, acc)
        issue(ci + 2, 0)
        drain(1)
        acc = compute(1, acc)
        return acc

    acc = lax.fori_loop(0, N_CHUNKS // 2, pair_body,
                        jnp.zeros((L,), jnp.float32))
    drain(0)

    accv[...] = acc
    pltpu.sync_copy(accv, out_hbm.at[b])


@jax.jit
def _sc_energy(x, j, ei, ej):
    mesh = plsc.VectorSubcoreMesh(core_axis_name="c", subcore_axis_name="s",
                                  num_cores=NC, num_subcores=NS)
    run = pl.kernel(
        _sc_body,
        out_type=jax.ShapeDtypeStruct((BATCH, L), jnp.float32),
        mesh=mesh,
        compiler_params=pltpu.CompilerParams(needs_layout_passes=False),
        scratch_types=[
            pltpu.VMEM((NUM_NODES,), jnp.float32),
            pltpu.VMEM((2, CHUNK), jnp.int32),
            pltpu.VMEM((2, CHUNK), jnp.int32),
            pltpu.VMEM((2, CHUNK), jnp.float32),
            pltpu.VMEM((L,), jnp.float32),
            pltpu.SemaphoreType.DMA,
            pltpu.SemaphoreType.DMA,
        ],
    )
    return run(x, j, ei, ej)


def _xh_body(x_ref, h_ref, out_ref):
    out_ref[...] = jnp.sum(x_ref[...] * h_ref[...], axis=1, keepdims=True)


@jax.jit
def _xh_matvec(x, h):
    return pl.pallas_call(
        _xh_body,
        out_shape=jax.ShapeDtypeStruct((BATCH, 1), jnp.float32),
    )(x, h.reshape(1, NUM_NODES))


def kernel(x, h, J, edge_idx_i, edge_idx_j):
    partials = _sc_energy(x, J,
                          edge_idx_i.astype(jnp.int32),
                          edge_idx_j.astype(jnp.int32))
    xh = _xh_matvec(x, h)
    return partials.sum(axis=-1) + xh[:, 0]


# speedup vs baseline: 14.6624x; 1.3282x over previous
import functools

import jax
import jax.numpy as jnp
from jax import lax
from jax.experimental import pallas as pl
from jax.experimental.pallas import tpu as pltpu
from jax.experimental.pallas import tpu_sc as plsc

NUM_NODES = 50000
NUM_EDGES = 1600000
BATCH = 32

NC = 2
NS = 16
L = 16

CHUNK = 6400
N_CHUNKS = NUM_EDGES // CHUNK
G_PER_CHUNK = CHUNK // L


def _sc_body(x_hbm, j_hbm, ei_hbm, ej_hbm, out_hbm,
             xb, eic, ejc, jc, accv, sem0, sem1):
    cid = lax.axis_index("c")
    sid = lax.axis_index("s")
    b = sid * NC + cid

    pltpu.sync_copy(x_hbm.at[b], xb)

    sems = (sem0, sem1)

    def issue(ci, slot):
        base = lax.rem(ci, N_CHUNKS) * CHUNK
        pltpu.async_copy(ei_hbm.at[pl.ds(base, CHUNK)], eic.at[slot], sems[slot])
        pltpu.async_copy(ej_hbm.at[pl.ds(base, CHUNK)], ejc.at[slot], sems[slot])
        pltpu.async_copy(j_hbm.at[pl.ds(base, CHUNK)], jc.at[slot], sems[slot])

    def drain(slot):
        pltpu.make_async_copy(ei_hbm.at[pl.ds(0, CHUNK)], eic.at[slot], sems[slot]).wait()
        pltpu.make_async_copy(ej_hbm.at[pl.ds(0, CHUNK)], ejc.at[slot], sems[slot]).wait()
        pltpu.make_async_copy(j_hbm.at[pl.ds(0, CHUNK)], jc.at[slot], sems[slot]).wait()

    def compute(slot, acc):
        @plsc.parallel_loop(0, CHUNK, L, unroll=8, carry=acc)
        def loop(off, acc):
            ii = eic[slot, pl.ds(off, L)]
            jj = ejc[slot, pl.ds(off, L)]
            vi = plsc.load_gather(xb, [ii])
            vj = plsc.load_gather(xb, [jj])
            jv = jc[slot, pl.ds(off, L)]
            return acc + jv * (vi * vj)

        return loop

    issue(0, 0)

    def pair_body(p, acc):
        ci = 2 * p
        issue(ci + 1, 1)
        drain(0)
        acc = compute(0, acc)
        issue(ci + 2, 0)
        drain(1)
        acc = compute(1, acc)
        return acc

    acc = lax.fori_loop(0, N_CHUNKS // 2, pair_body,
                        jnp.zeros((L,), jnp.float32))
    drain(0)

    accv[...] = acc
    pltpu.sync_copy(accv, out_hbm.at[b])


@jax.jit
def _sc_energy(x, j, ei, ej):
    mesh = plsc.VectorSubcoreMesh(core_axis_name="c", subcore_axis_name="s",
                                  num_cores=NC, num_subcores=NS)
    run = pl.kernel(
        _sc_body,
        out_type=jax.ShapeDtypeStruct((BATCH, L), jnp.float32),
        mesh=mesh,
        compiler_params=pltpu.CompilerParams(needs_layout_passes=False),
        scratch_types=[
            pltpu.VMEM((NUM_NODES,), jnp.float32),
            pltpu.VMEM((2, CHUNK), jnp.int32),
            pltpu.VMEM((2, CHUNK), jnp.int32),
            pltpu.VMEM((2, CHUNK), jnp.float32),
            pltpu.VMEM((L,), jnp.float32),
            pltpu.SemaphoreType.DMA,
            pltpu.SemaphoreType.DMA,
        ],
    )
    return run(x, j, ei, ej)


def _xh_body(x_ref, h_ref, out_ref):
    out_ref[...] = jnp.sum(x_ref[...] * h_ref[...], axis=1, keepdims=True)


@jax.jit
def _xh_matvec(x, h):
    return pl.pallas_call(
        _xh_body,
        out_shape=jax.ShapeDtypeStruct((BATCH, 1), jnp.float32),
    )(x, h.reshape(1, NUM_NODES))


def kernel(x, h, J, edge_idx_i, edge_idx_j):
    partials = _sc_energy(x, J,
                          edge_idx_i.astype(jnp.int32),
                          edge_idx_j.astype(jnp.int32))
    xh = _xh_matvec(x, h)
    return partials.sum(axis=-1) + xh[:, 0]


# speedup vs baseline: 21.8780x; 1.4921x over previous
import functools

import jax
import jax.numpy as jnp
from jax import lax
from jax.experimental import pallas as pl
from jax.experimental.pallas import tpu as pltpu
from jax.experimental.pallas import tpu_sc as plsc

NUM_NODES = 50000
NUM_EDGES = 1600000
BATCH = 32

NC = 2
NS = 16
L = 16

EDGE_HALF = NUM_EDGES // 2
CHUNK = 3200
N_CHUNKS = EDGE_HALF // CHUNK


def _sc_body(x_hbm, j_hbm, ei_hbm, ej_hbm, out_hbm,
             xb0, xb1, eic, ejc, jc, accv, sem0, sem1):
    cid = lax.axis_index("c")
    sid = lax.axis_index("s")
    wid = sid * NC + cid
    pair = lax.rem(wid, NS)
    half = wid // NS
    b0 = 2 * pair
    edge_base = half * EDGE_HALF

    pltpu.sync_copy(x_hbm.at[b0], xb0)
    pltpu.sync_copy(x_hbm.at[b0 + 1], xb1)

    sems = (sem0, sem1)

    def issue(ci, slot):
        base = edge_base + lax.rem(ci, N_CHUNKS) * CHUNK
        pltpu.async_copy(ei_hbm.at[pl.ds(base, CHUNK)], eic.at[slot], sems[slot])
        pltpu.async_copy(ej_hbm.at[pl.ds(base, CHUNK)], ejc.at[slot], sems[slot])
        pltpu.async_copy(j_hbm.at[pl.ds(base, CHUNK)], jc.at[slot], sems[slot])

    def drain(slot):
        pltpu.make_async_copy(ei_hbm.at[pl.ds(0, CHUNK)], eic.at[slot], sems[slot]).wait()
        pltpu.make_async_copy(ej_hbm.at[pl.ds(0, CHUNK)], ejc.at[slot], sems[slot]).wait()
        pltpu.make_async_copy(j_hbm.at[pl.ds(0, CHUNK)], jc.at[slot], sems[slot]).wait()

    def compute(slot, acc):
        @plsc.parallel_loop(0, CHUNK, L, unroll=8, carry=acc)
        def loop(off, acc):
            acc0, acc1 = acc
            ii = eic[slot, pl.ds(off, L)]
            jj = ejc[slot, pl.ds(off, L)]
            jv = jc[slot, pl.ds(off, L)]
            vi0 = plsc.load_gather(xb0, [ii])
            vj0 = plsc.load_gather(xb0, [jj])
            vi1 = plsc.load_gather(xb1, [ii])
            vj1 = plsc.load_gather(xb1, [jj])
            return (acc0 + jv * (vi0 * vj0), acc1 + jv * (vi1 * vj1))

        return loop

    issue(0, 0)

    def pair_body(p, acc):
        ci = 2 * p
        issue(ci + 1, 1)
        drain(0)
        acc = compute(0, acc)
        issue(ci + 2, 0)
        drain(1)
        acc = compute(1, acc)
        return acc

    zero = jnp.zeros((L,), jnp.float32)
    acc0, acc1 = lax.fori_loop(0, N_CHUNKS // 2, pair_body, (zero, zero))
    drain(0)

    accv[0, :] = acc0
    accv[1, :] = acc1
    pltpu.sync_copy(accv.at[0], out_hbm.at[b0, half])
    pltpu.sync_copy(accv.at[1], out_hbm.at[b0 + 1, half])


@jax.jit
def _sc_energy(x, j, ei, ej):
    mesh = plsc.VectorSubcoreMesh(core_axis_name="c", subcore_axis_name="s",
                                  num_cores=NC, num_subcores=NS)
    run = pl.kernel(
        _sc_body,
        out_type=jax.ShapeDtypeStruct((BATCH, 2, L), jnp.float32),
        mesh=mesh,
        compiler_params=pltpu.CompilerParams(needs_layout_passes=False),
        scratch_types=[
            pltpu.VMEM((NUM_NODES,), jnp.float32),
            pltpu.VMEM((NUM_NODES,), jnp.float32),
            pltpu.VMEM((2, CHUNK), jnp.int32),
            pltpu.VMEM((2, CHUNK), jnp.int32),
            pltpu.VMEM((2, CHUNK), jnp.float32),
            pltpu.VMEM((2, L), jnp.float32),
            pltpu.SemaphoreType.DMA,
            pltpu.SemaphoreType.DMA,
        ],
    )
    return run(x, j, ei, ej)


def _xh_body(x_ref, h_ref, out_ref):
    out_ref[...] = jnp.sum(x_ref[...] * h_ref[...], axis=1, keepdims=True)


@jax.jit
def _xh_matvec(x, h):
    return pl.pallas_call(
        _xh_body,
        out_shape=jax.ShapeDtypeStruct((BATCH, 1), jnp.float32),
    )(x, h.reshape(1, NUM_NODES))


def kernel(x, h, J, edge_idx_i, edge_idx_j):
    partials = _sc_energy(x, J,
                          edge_idx_i.astype(jnp.int32),
                          edge_idx_j.astype(jnp.int32))
    xh = _xh_matvec(x, h)
    return partials.sum(axis=(-2, -1)) + xh[:, 0]


# speedup vs baseline: 30.1505x; 1.3781x over previous
import functools

import jax
import jax.numpy as jnp
from jax import lax
from jax.experimental import pallas as pl
from jax.experimental.pallas import tpu as pltpu
from jax.experimental.pallas import tpu_sc as plsc

NUM_NODES = 50000
NUM_EDGES = 1600000
BATCH = 32

NC = 2
NS = 16
L = 16

N_QUARTERS = 4
N_GROUPS = 8
EDGE_Q = NUM_EDGES // N_QUARTERS
CHUNK = 3200
N_CHUNKS = EDGE_Q // CHUNK

_ILV = plsc.PackFormat.INTERLEAVED


def _sc_body(xp_hbm, j_hbm, ei_hbm, ej_hbm, out_hbm,
             xpa, xpb, eic, ejc, jc, accv, sem0, sem1):
    cid = lax.axis_index("c")
    sid = lax.axis_index("s")
    wid = sid * NC + cid
    g = lax.rem(wid, N_GROUPS)
    q = wid // N_GROUPS
    edge_base = q * EDGE_Q

    pltpu.sync_copy(xp_hbm.at[2 * g], xpa)
    pltpu.sync_copy(xp_hbm.at[2 * g + 1], xpb)

    sems = (sem0, sem1)

    def issue(ci, slot):
        base = edge_base + ci * CHUNK
        pltpu.async_copy(ei_hbm.at[pl.ds(base, CHUNK)], eic.at[slot], sems[slot])
        pltpu.async_copy(ej_hbm.at[pl.ds(base, CHUNK)], ejc.at[slot], sems[slot])
        pltpu.async_copy(j_hbm.at[pl.ds(base, CHUNK)], jc.at[slot], sems[slot])

    def drain(slot):
        pltpu.make_async_copy(ei_hbm.at[pl.ds(0, CHUNK)], eic.at[slot], sems[slot]).wait()
        pltpu.make_async_copy(ej_hbm.at[pl.ds(0, CHUNK)], ejc.at[slot], sems[slot]).wait()
        pltpu.make_async_copy(j_hbm.at[pl.ds(0, CHUNK)], jc.at[slot], sems[slot]).wait()

    def compute(slot, acc):
        @plsc.parallel_loop(0, CHUNK, L, unroll=8, carry=acc)
        def loop(off, acc):
            a0, a1, a2, a3 = acc
            ii = eic[slot, pl.ds(off, L)]
            jj = ejc[slot, pl.ds(off, L)]
            jv = jc[slot, pl.ds(off, L)]
            wia = plsc.load_gather(xpa, [ii])
            wja = plsc.load_gather(xpa, [jj])
            wib = plsc.load_gather(xpb, [ii])
            wjb = plsc.load_gather(xpb, [jj])
            i0, i1 = plsc.unpack(plsc.bitcast(wia, jnp.bfloat16), format=_ILV)
            j0, j1 = plsc.unpack(plsc.bitcast(wja, jnp.bfloat16), format=_ILV)
            i2, i3 = plsc.unpack(plsc.bitcast(wib, jnp.bfloat16), format=_ILV)
            j2, j3 = plsc.unpack(plsc.bitcast(wjb, jnp.bfloat16), format=_ILV)
            return (a0 + jv * (i0 * j0), a1 + jv * (i1 * j1),
                    a2 + jv * (i2 * j2), a3 + jv * (i3 * j3))

        return loop

    zero = jnp.zeros((L,), jnp.float32)
    acc = (zero, zero, zero, zero)

    issue(0, 0)

    def pair_body(p, acc):
        ci = 2 * p
        issue(ci + 1, 1)
        drain(0)
        acc = compute(0, acc)
        issue(ci + 2, 0)
        drain(1)
        acc = compute(1, acc)
        return acc

    acc = lax.fori_loop(0, (N_CHUNKS - 1) // 2, pair_body, acc)
    drain(0)
    acc = compute(0, acc)

    for k in range(4):
        accv[k, :] = acc[k]
    for k in range(4):
        pltpu.sync_copy(accv.at[k], out_hbm.at[g, k, q])


@jax.jit
def _sc_energy(xp, j, ei, ej):
    mesh = plsc.VectorSubcoreMesh(core_axis_name="c", subcore_axis_name="s",
                                  num_cores=NC, num_subcores=NS)
    run = pl.kernel(
        _sc_body,
        out_type=jax.ShapeDtypeStruct((N_GROUPS, 4, N_QUARTERS, L),
                                      jnp.float32),
        mesh=mesh,
        compiler_params=pltpu.CompilerParams(needs_layout_passes=False),
        scratch_types=[
            pltpu.VMEM((NUM_NODES,), jnp.float32),
            pltpu.VMEM((NUM_NODES,), jnp.float32),
            pltpu.VMEM((2, CHUNK), jnp.int32),
            pltpu.VMEM((2, CHUNK), jnp.int32),
            pltpu.VMEM((2, CHUNK), jnp.float32),
            pltpu.VMEM((4, L), jnp.float32),
            pltpu.SemaphoreType.DMA,
            pltpu.SemaphoreType.DMA,
        ],
    )
    return run(xp, j, ei, ej)


def _xh_body(x_ref, h_ref, out_ref):
    out_ref[...] = jnp.sum(x_ref[...] * h_ref[...], axis=1, keepdims=True)


@jax.jit
def _xh_matvec(x, h):
    return pl.pallas_call(
        _xh_body,
        out_shape=jax.ShapeDtypeStruct((BATCH, 1), jnp.float32),
    )(x, h.reshape(1, NUM_NODES))


def kernel(x, h, J, edge_idx_i, edge_idx_j):
    xr = x.astype(jnp.bfloat16).reshape(BATCH // 2, 2, NUM_NODES)
    xr = jnp.swapaxes(xr, 1, 2)
    xp = lax.bitcast_convert_type(xr, jnp.float32)

    partials = _sc_energy(xp, J,
                          edge_idx_i.astype(jnp.int32),
                          edge_idx_j.astype(jnp.int32))
    xh = _xh_matvec(x, h)
    return partials.reshape(BATCH, N_QUARTERS * L).sum(axis=-1) + xh[:, 0]


# speedup vs baseline: 30.5481x; 1.0132x over previous
import functools

import jax
import jax.numpy as jnp
from jax import lax
from jax.experimental import pallas as pl
from jax.experimental.pallas import tpu as pltpu
from jax.experimental.pallas import tpu_sc as plsc

NUM_NODES = 50000
NUM_EDGES = 1600000
BATCH = 32

NC = 2
NS = 16
L = 16

N_QUARTERS = 4
N_GROUPS = 8
EDGE_Q = NUM_EDGES // N_QUARTERS
CHUNK = 3200
N_CHUNKS = EDGE_Q // CHUNK

_ILV = plsc.PackFormat.INTERLEAVED


def _sc_body(xp_hbm, j_hbm, ep_hbm, out_hbm,
             xpa, xpb, epc, jc, accv, sem0, sem1):
    cid = lax.axis_index("c")
    sid = lax.axis_index("s")
    wid = sid * NC + cid
    g = lax.rem(wid, N_GROUPS)
    q = wid // N_GROUPS
    edge_base = q * EDGE_Q

    pltpu.sync_copy(xp_hbm.at[2 * g], xpa)
    pltpu.sync_copy(xp_hbm.at[2 * g + 1], xpb)

    sems = (sem0, sem1)

    def issue(ci, slot):
        base = edge_base + ci * CHUNK
        pltpu.async_copy(ep_hbm.at[pl.ds(base, CHUNK)], epc.at[slot], sems[slot])
        pltpu.async_copy(j_hbm.at[pl.ds(base, CHUNK)], jc.at[slot], sems[slot])

    def drain(slot):
        pltpu.make_async_copy(ep_hbm.at[pl.ds(0, CHUNK)], epc.at[slot], sems[slot]).wait()
        pltpu.make_async_copy(j_hbm.at[pl.ds(0, CHUNK)], jc.at[slot], sems[slot]).wait()

    def compute(slot, acc):
        @plsc.parallel_loop(0, CHUNK, L, unroll=8, carry=acc)
        def loop(off, acc):
            a0, a1, a2, a3 = acc
            w = epc[slot, pl.ds(off, L)]
            jv = jc[slot, pl.ds(off, L)]
            ii = w & 0xFFFF
            jj = lax.shift_right_logical(w, 16)
            wia = plsc.load_gather(xpa, [ii])
            wja = plsc.load_gather(xpa, [jj])
            wib = plsc.load_gather(xpb, [ii])
            wjb = plsc.load_gather(xpb, [jj])
            i0, i1 = plsc.unpack(plsc.bitcast(wia, jnp.bfloat16), format=_ILV)
            j0, j1 = plsc.unpack(plsc.bitcast(wja, jnp.bfloat16), format=_ILV)
            i2, i3 = plsc.unpack(plsc.bitcast(wib, jnp.bfloat16), format=_ILV)
            j2, j3 = plsc.unpack(plsc.bitcast(wjb, jnp.bfloat16), format=_ILV)
            return (a0 + jv * (i0 * j0), a1 + jv * (i1 * j1),
                    a2 + jv * (i2 * j2), a3 + jv * (i3 * j3))

        return loop

    zero = jnp.zeros((L,), jnp.float32)
    acc = (zero, zero, zero, zero)

    issue(0, 0)

    def pair_body(p, acc):
        ci = 2 * p
        issue(ci + 1, 1)
        drain(0)
        acc = compute(0, acc)
        issue(ci + 2, 0)
        drain(1)
        acc = compute(1, acc)
        return acc

    acc = lax.fori_loop(0, (N_CHUNKS - 1) // 2, pair_body, acc)
    drain(0)
    acc = compute(0, acc)

    for k in range(4):
        accv[k, :] = acc[k]
    for k in range(4):
        pltpu.sync_copy(accv.at[k], out_hbm.at[g, k, q])


@jax.jit
def _sc_energy(xp, j, ep):
    mesh = plsc.VectorSubcoreMesh(core_axis_name="c", subcore_axis_name="s",
                                  num_cores=NC, num_subcores=NS)
    run = pl.kernel(
        _sc_body,
        out_type=jax.ShapeDtypeStruct((N_GROUPS, 4, N_QUARTERS, L),
                                      jnp.float32),
        mesh=mesh,
        compiler_params=pltpu.CompilerParams(needs_layout_passes=False),
        scratch_types=[
            pltpu.VMEM((NUM_NODES,), jnp.float32),
            pltpu.VMEM((NUM_NODES,), jnp.float32),
            pltpu.VMEM((2, CHUNK), jnp.int32),
            pltpu.VMEM((2, CHUNK), jnp.float32),
            pltpu.VMEM((4, L), jnp.float32),
            pltpu.SemaphoreType.DMA,
            pltpu.SemaphoreType.DMA,
        ],
    )
    return run(xp, j, ep)


def _xh_body(x_ref, h_ref, out_ref):
    out_ref[...] = jnp.sum(x_ref[...] * h_ref[...], axis=1, keepdims=True)


@jax.jit
def _xh_matvec(x, h):
    return pl.pallas_call(
        _xh_body,
        out_shape=jax.ShapeDtypeStruct((BATCH, 1), jnp.float32),
    )(x, h.reshape(1, NUM_NODES))


def kernel(x, h, J, edge_idx_i, edge_idx_j):
    xr = x.astype(jnp.bfloat16).reshape(BATCH // 2, 2, NUM_NODES)
    xr = jnp.swapaxes(xr, 1, 2)
    xp = lax.bitcast_convert_type(xr, jnp.float32)

    ei = edge_idx_i.astype(jnp.int32)
    ej = edge_idx_j.astype(jnp.int32)
    ep = ei | (ej << 16)

    partials = _sc_energy(xp, J, ep)
    xh = _xh_matvec(x, h)
    return partials.reshape(BATCH, N_QUARTERS * L).sum(axis=-1) + xh[:, 0]
